# baseline (speedup 1.0000x reference)
import jax
import jax.numpy as jnp
from jax.experimental import pallas as pl
from jax.experimental.pallas import tpu as pltpu

_B, _M, _C = 2, 4096, 16
_W = _C + _M
_R = 256
_G = 512
_NT = _M // _R


def _loss_body(scores_ref, lt_ref, clen_ref, cid_rows_ref, cid_pad_ref,
               out_ref, csize_ref, sdiag_ref):
    b = pl.program_id(0)
    t = pl.program_id(1)

    @pl.when(t == 0)
    def _build():
        cid_col = cid_pad_ref[0]
        gid = jax.lax.broadcasted_iota(jnp.int32, (_W, _G), 1)
        hit = gid == cid_col.reshape(_W, 1)
        csize_ref[...] = jnp.sum(jnp.where(hit, 1.0, 0.0), axis=0, keepdims=True)

    rr = jax.lax.broadcasted_iota(jnp.int32, (_R, _R), 0)
    cc = jax.lax.broadcasted_iota(jnp.int32, (_R, _R), 1)
    eye_rr = rr == cc
    for k in range(_NT):
        @pl.when(t == k)
        def _extract(k=k):
            win = scores_ref[0, :, (_C + k * _R):(_C + (k + 1) * _R)]
            sdiag_ref[...] = jnp.sum(jnp.where(eye_rr, win, 0.0), axis=1,
                                     keepdims=True)

    s = scores_ref[0]
    lt = lt_ref[0]
    clen = clen_ref[0][:, 0:1]
    cid_r = cid_rows_ref[0][:, 0:1]
    cid_p = cid_pad_ref[0]

    m = jnp.max(s, axis=1, keepdims=True)
    e = jnp.exp(s - m)
    sum_e = jnp.sum(e, axis=1, keepdims=True)
    same = cid_p == cid_r
    sum_same_e = jnp.sum(jnp.where(same, e, 0.0), axis=1, keepdims=True)

    e_diag = jnp.exp(sdiag_ref[...] - m)
    sum_mates_e = jnp.maximum(sum_same_e - e_diag, 0.0)

    gr = jax.lax.broadcasted_iota(jnp.int32, (_R, _G), 1)
    row_oh = gr == cid_r
    cnt_same = jnp.sum(jnp.where(row_oh, csize_ref[...], 0.0), axis=1,
                       keepdims=True)

    c16 = jax.lax.broadcasted_iota(jnp.int32, (_R, _C), 1)
    e_l = e[:, :_C]
    link_valid = c16 < clen
    sum_inv_l = jnp.sum(jnp.where(link_valid, 0.0, e_l), axis=1, keepdims=True)
    gold_l = jnp.logical_and(lt != 0, link_valid)
    sum_gold_l = jnp.sum(jnp.where(gold_l, e_l, 0.0), axis=1, keepdims=True)
    cnt_gold_l = jnp.sum(jnp.where(gold_l, 1.0, 0.0), axis=1, keepdims=True)

    num_found = (cnt_same - 1.0) + cnt_gold_l
    self_f = jnp.where(num_found == 0.0, 1.0, 0.0)

    sum_all = sum_e - sum_inv_l
    sum_gold = sum_mates_e + self_f * e_diag + sum_gold_l

    contrib = jnp.sum(jnp.log(sum_all) - jnp.log(sum_gold), axis=0, keepdims=True)

    @pl.when(jnp.logical_and(b == 0, t == 0))
    def _init():
        out_ref[...] = jnp.zeros((1, 1), jnp.float32)

    out_ref[...] += contrib


@jax.jit
def kernel(scores, linker_targets, candidate_lengths, cluster_ids):
    B, M, W = scores.shape
    C = W - M
    clen = jnp.broadcast_to(candidate_lengths[:, :, None], (B, M, 128))
    cid_r = jnp.broadcast_to(cluster_ids[:, :, None], (B, M, 128))
    cid_p = jnp.concatenate(
        [jnp.full((B, 1, C), -1, jnp.int32), cluster_ids.reshape(B, 1, M)],
        axis=-1,
    )

    grid = (B, M // _R)
    out = pl.pallas_call(
        _loss_body,
        grid=grid,
        in_specs=[
            pl.BlockSpec((1, _R, W), lambda b, t: (b, t, 0)),
            pl.BlockSpec((1, _R, C), lambda b, t: (b, t, 0)),
            pl.BlockSpec((1, _R, 128), lambda b, t: (b, t, 0)),
            pl.BlockSpec((1, _R, 128), lambda b, t: (b, t, 0)),
            pl.BlockSpec((1, 1, W), lambda b, t: (b, 0, 0)),
        ],
        out_specs=pl.BlockSpec((1, 1), lambda b, t: (0, 0)),
        out_shape=jax.ShapeDtypeStruct((1, 1), jnp.float32),
        scratch_shapes=[
            pltpu.VMEM((1, _G), jnp.float32),
            pltpu.VMEM((_R, 1), jnp.float32),
        ],
        compiler_params=pltpu.CompilerParams(
            dimension_semantics=("arbitrary", "arbitrary"),
        ),
    )(scores, linker_targets, clen, cid_r, cid_p)
    return out[0, 0]


# speedup vs baseline: 2.6747x; 2.6747x over previous
import jax
import jax.numpy as jnp
from jax.experimental import pallas as pl
from jax.experimental.pallas import tpu as pltpu

_B, _M, _C = 2, 4096, 16
_W = _C + _M
_R = 256
_G = 512
_NT = _M // _R


def _loss_body(scores_ref, lt_ref, clen_ref, cid_tile_ref, cid_full_ref,
               cid_col_ref, out_ref, csize_ref, sdiag_ref):
    b = pl.program_id(0)
    t = pl.program_id(1)

    @pl.when(t == 0)
    def _build():
        cid_all = cid_full_ref[0]
        gid = jax.lax.broadcasted_iota(jnp.int32, (_G, _M), 0)
        hit = gid == cid_all
        csize_ref[...] = jnp.sum(jnp.where(hit, 1.0, 0.0), axis=1, keepdims=True)

    rr = jax.lax.broadcasted_iota(jnp.int32, (_R, _R), 0)
    cc = jax.lax.broadcasted_iota(jnp.int32, (_R, _R), 1)
    eye_rr = rr == cc
    for k in range(_NT):
        @pl.when(t == k)
        def _extract(k=k):
            win = scores_ref[0, (_C + k * _R):(_C + (k + 1) * _R), :]
            sdiag_ref[...] = jnp.sum(jnp.where(eye_rr, win, 0.0), axis=0,
                                     keepdims=True)

    s = scores_ref[0]
    lt = lt_ref[0]
    clen = clen_ref[0]
    cid_row = cid_tile_ref[0]
    cid_col = cid_col_ref[0]

    m = jnp.max(s, axis=0, keepdims=True)
    e = jnp.exp(s - m)
    sum_e = jnp.sum(e, axis=0, keepdims=True)

    e_c = e[_C:, :]
    same = cid_col == cid_row
    sum_same_e = jnp.sum(jnp.where(same, e_c, 0.0), axis=0, keepdims=True)

    e_diag = jnp.exp(sdiag_ref[...] - m)
    sum_mates_e = jnp.maximum(sum_same_e - e_diag, 0.0)

    gidr = jax.lax.broadcasted_iota(jnp.int32, (_G, _R), 0)
    row_oh = gidr == cid_row
    cnt_same = jnp.sum(jnp.where(row_oh, csize_ref[...], 0.0), axis=0,
                       keepdims=True)

    c16 = jax.lax.broadcasted_iota(jnp.int32, (_C, _R), 0)
    e_l = e[:_C, :]
    link_valid = c16 < clen
    sum_inv_l = jnp.sum(jnp.where(link_valid, 0.0, e_l), axis=0, keepdims=True)
    gold_l = jnp.logical_and(lt != 0, link_valid)
    sum_gold_l = jnp.sum(jnp.where(gold_l, e_l, 0.0), axis=0, keepdims=True)
    cnt_gold_l = jnp.sum(jnp.where(gold_l, 1.0, 0.0), axis=0, keepdims=True)

    num_found = (cnt_same - 1.0) + cnt_gold_l
    self_f = jnp.where(num_found == 0.0, 1.0, 0.0)

    sum_all = sum_e - sum_inv_l
    sum_gold = sum_mates_e + self_f * e_diag + sum_gold_l

    contrib = jnp.sum(jnp.log(sum_all) - jnp.log(sum_gold), axis=1, keepdims=True)

    @pl.when(jnp.logical_and(b == 0, t == 0))
    def _init():
        out_ref[...] = jnp.zeros((1, 1), jnp.float32)

    out_ref[...] += contrib


@jax.jit
def kernel(scores, linker_targets, candidate_lengths, cluster_ids):
    B, M, W = scores.shape
    C = W - M
    scores_t = jnp.transpose(scores, (0, 2, 1))
    lt_t = jnp.transpose(linker_targets, (0, 2, 1))
    clen2 = candidate_lengths.reshape(B, 1, M)
    cid2 = cluster_ids.reshape(B, 1, M)
    cid_col = cluster_ids.reshape(B, M, 1)

    grid = (B, M // _R)
    out = pl.pallas_call(
        _loss_body,
        grid=grid,
        in_specs=[
            pl.BlockSpec((1, W, _R), lambda b, t: (b, 0, t)),
            pl.BlockSpec((1, C, _R), lambda b, t: (b, 0, t)),
            pl.BlockSpec((1, 1, _R), lambda b, t: (b, 0, t)),
            pl.BlockSpec((1, 1, _R), lambda b, t: (b, 0, t)),
            pl.BlockSpec((1, 1, M), lambda b, t: (b, 0, 0)),
            pl.BlockSpec((1, M, 1), lambda b, t: (b, 0, 0)),
        ],
        out_specs=pl.BlockSpec((1, 1), lambda b, t: (0, 0)),
        out_shape=jax.ShapeDtypeStruct((1, 1), jnp.float32),
        scratch_shapes=[
            pltpu.VMEM((_G, 1), jnp.float32),
            pltpu.VMEM((1, _R), jnp.float32),
        ],
        compiler_params=pltpu.CompilerParams(
            dimension_semantics=("arbitrary", "arbitrary"),
        ),
    )(scores_t, lt_t, clen2, cid2, cid2, cid_col)
    return out[0, 0]
